# Initial kernel scaffold; baseline (speedup 1.0000x reference)
import jax
import jax.numpy as jnp
from jax.experimental import pallas as pl


def kernel(x, lmda, perm):
    raise NotImplementedError("write your pallas kernel here")


# speedup vs baseline: 3.6249x; 3.6249x over previous
import functools

import jax
import jax.numpy as jnp
from jax import lax
from jax.experimental import pallas as pl
from jax.experimental.pallas import tpu as pltpu
from jax.experimental.pallas import tpu_sc as plsc

L = 16
NBITS = 11
NBUCK = 1 << NBITS
DMASK = NBUCK - 1
TOPBIT = -2147483648


def _wid():
    return lax.axis_index("s") * 2 + lax.axis_index("c")


def _sort_rows_body(n, rpw, x_hbm, vals_hbm, idx_hbm, xbuf, k0, p0, k1, p1, hist):
    nv = n // L
    nhv = NBUCK // L
    w = _wid()
    iota = lax.iota(jnp.int32, L)

    def row_body(j, _):
        row = w * rpw + j
        pltpu.sync_copy(x_hbm.at[row], xbuf)

        def conv_in(i, _):
            s = pl.ds(i * L, L)
            bits = lax.bitcast_convert_type(xbuf[s], jnp.int32)
            xm = lax.shift_right_arithmetic(bits, 31) | TOPBIT
            k0[s] = bits ^ xm
            p0[s] = iota + i * L
            return 0

        lax.fori_loop(0, nv, conv_in, 0)

        bufs = [(k0, p0, k1, p1), (k1, p1, k0, p0), (k0, p0, k1, p1)]
        for pss, (ksrc, psrc, kdst, pdst) in enumerate(bufs):
            shift = pss * NBITS

            def zero_body(i, _):
                hist[pl.ds(i * L, L)] = jnp.zeros((L,), jnp.int32)
                return 0

            lax.fori_loop(0, nhv, zero_body, 0)

            def hist_body(i, _):
                s = pl.ds(i * L, L)
                d = lax.shift_right_logical(ksrc[s], shift) & DMASK
                occ, last = plsc.scan_count(d)
                plsc.addupdate_scatter(hist, [d], occ, mask=last)
                return 0

            lax.fori_loop(0, nv, hist_body, 0)

            def scan_body(i, carry):
                s = pl.ds(i * L, L)
                v = hist[s]
                cs = plsc.cumsum(v)
                hist[s] = cs - v + carry
                return carry + jnp.sum(v)

            lax.fori_loop(0, nhv, scan_body, jnp.int32(0))

            def scat_body(i, _):
                s = pl.ds(i * L, L)
                k = ksrc[s]
                p = psrc[s]
                d = lax.shift_right_logical(k, shift) & DMASK
                occ, last = plsc.scan_count(d)
                base = plsc.load_gather(hist, [d])
                off = base + occ - 1
                plsc.store_scatter(kdst, [off], k)
                plsc.store_scatter(pdst, [off], p)
                plsc.addupdate_scatter(hist, [d], occ, mask=last)
                return 0

            lax.fori_loop(0, nv, scat_body, 0)

        def conv_out(i, _):
            s = pl.ds(i * L, L)
            u = k1[s]
            xm = ~lax.shift_right_arithmetic(u, 31) | TOPBIT
            xbuf[s] = lax.bitcast_convert_type(u ^ xm, jnp.float32)
            return 0

        lax.fori_loop(0, nv, conv_out, 0)
        pltpu.sync_copy(xbuf, vals_hbm.at[row])
        pltpu.sync_copy(p1, idx_hbm.at[row])
        return 0

    lax.fori_loop(0, rpw, row_body, 0)


def _mix_rows_body(n, rpw, nch, lam_hbm, pm_hbm, vals_hbm, idx_hbm, out_hbm,
                   lbuf, pbuf, va, vb, ib, ob):
    nv = n // L
    w = _wid()
    b = w // 2
    half = (w % 2) * rpw
    iota = lax.iota(jnp.int32, L)

    pltpu.sync_copy(lam_hbm, lbuf)
    pltpu.sync_copy(pm_hbm, pbuf)
    lam = jnp.sum(jnp.where(iota == b, lbuf[...], 0.0))
    pb = jnp.sum(jnp.where(iota == b, pbuf[...], 0))
    lamv = jnp.full((L,), lam, jnp.float32)
    one_m = jnp.full((L,), 1.0, jnp.float32) - lamv

    def row_body(j, _):
        row = w * rpw + j
        prow = pb * nch + half + j
        pltpu.sync_copy(vals_hbm.at[row], va)
        pltpu.sync_copy(vals_hbm.at[prow], vb)
        pltpu.sync_copy(idx_hbm.at[row], ib)

        def mix_body(i, _):
            s = pl.ds(i * L, L)
            mixed = lamv * va[s] + one_m * vb[s]
            plsc.store_scatter(ob, [ib[s]], mixed)
            return 0

        lax.fori_loop(0, nv, mix_body, 0)
        pltpu.sync_copy(ob, out_hbm.at[row])
        return 0

    lax.fori_loop(0, rpw, row_body, 0)


@jax.jit
def kernel(x, lmda, perm):
    bv, cv, hv, wv = x.shape
    n = hv * wv
    r = bv * cv
    nw = 32
    assert r % nw == 0 and n % L == 0
    rpw = r // nw

    xv = x.reshape(r, n)
    lam = lmda.reshape(bv).astype(jnp.float32)
    pm = perm.astype(jnp.int32)

    mesh = plsc.VectorSubcoreMesh(core_axis_name="c", subcore_axis_name="s")

    sort_call = pl.kernel(
        functools.partial(_sort_rows_body, n, rpw),
        out_type=[
            jax.ShapeDtypeStruct((r, n), jnp.float32),
            jax.ShapeDtypeStruct((r, n), jnp.int32),
        ],
        mesh=mesh,
        scratch_types=[
            pltpu.VMEM((n,), jnp.float32),
            pltpu.VMEM((n,), jnp.int32),
            pltpu.VMEM((n,), jnp.int32),
            pltpu.VMEM((n,), jnp.int32),
            pltpu.VMEM((n,), jnp.int32),
            pltpu.VMEM((NBUCK,), jnp.int32),
        ],
        compiler_params=pltpu.CompilerParams(needs_layout_passes=False),
    )
    vals, idxs = sort_call(xv)

    mix_call = pl.kernel(
        functools.partial(_mix_rows_body, n, rpw, cv),
        out_type=jax.ShapeDtypeStruct((r, n), jnp.float32),
        mesh=mesh,
        scratch_types=[
            pltpu.VMEM((bv,), jnp.float32),
            pltpu.VMEM((bv,), jnp.int32),
            pltpu.VMEM((n,), jnp.float32),
            pltpu.VMEM((n,), jnp.float32),
            pltpu.VMEM((n,), jnp.int32),
            pltpu.VMEM((n,), jnp.float32),
        ],
        compiler_params=pltpu.CompilerParams(needs_layout_passes=False),
    )
    out = mix_call(lam, pm, vals, idxs)
    return out.reshape(bv, cv, hv, wv)


# speedup vs baseline: 4.6270x; 1.2764x over previous
import functools

import jax
import jax.numpy as jnp
from jax import lax
from jax.experimental import pallas as pl
from jax.experimental.pallas import tpu as pltpu
from jax.experimental.pallas import tpu_sc as plsc

L = 16
NBITS = 8
NBUCK = 1 << NBITS
DMASK = NBUCK - 1
NPASS = 4
TOPBIT = -2147483648


def _wid():
    return lax.axis_index("s") * 2 + lax.axis_index("c")


def _sort_rows_body(n, rpw, x_hbm, vals_hbm, idx_hbm, xbuf, k0, p0, k1, p1, hist):
    nv = n // L
    w = _wid()
    iota = lax.iota(jnp.int32, L)

    def row_body(j, _):
        row = w * rpw + j
        pltpu.sync_copy(x_hbm.at[row], xbuf)

        def zero_body(i, _):
            hist[pl.ds(i * L, L)] = jnp.zeros((L,), jnp.int32)
            return 0

        lax.fori_loop(0, NPASS * NBUCK // L, zero_body, 0)

        def prep_body(i, _):
            s = pl.ds(i * L, L)
            bits = lax.bitcast_convert_type(xbuf[s], jnp.int32)
            xm = lax.shift_right_arithmetic(bits, 31) | TOPBIT
            k = bits ^ xm
            k0[s] = k
            p0[s] = iota + i * L
            for pss in range(NPASS):
                d = (lax.shift_right_logical(k, pss * NBITS) & DMASK) + pss * NBUCK
                occ, last = plsc.scan_count(d)
                plsc.addupdate_scatter(hist, [d], occ, mask=last)
            return 0

        lax.fori_loop(0, nv, prep_body, 0)

        def scan_body(i, carry):
            s = pl.ds(i * L, L)
            v = hist[s]
            cs = plsc.cumsum(v)
            hist[s] = cs - v + carry
            new = carry + jnp.sum(v)
            return jnp.where((i + 1) % (NBUCK // L) == 0, 0, new)

        lax.fori_loop(0, NPASS * NBUCK // L, scan_body, jnp.int32(0))

        bufs = [(k0, p0, k1, p1), (k1, p1, k0, p0),
                (k0, p0, k1, p1), (k1, p1, k0, p0)]
        for pss, (ksrc, psrc, kdst, pdst) in enumerate(bufs):
            last_pass = pss == NPASS - 1

            def scat_one(i):
                s = pl.ds(i * L, L)
                k = ksrc[s]
                p = psrc[s]
                d = (lax.shift_right_logical(k, pss * NBITS) & DMASK) + pss * NBUCK
                occ, last = plsc.scan_count(d)
                base = plsc.load_gather(hist, [d])
                off = base + occ - 1
                if last_pass:
                    xm = ~lax.shift_right_arithmetic(k, 31) | TOPBIT
                    f = lax.bitcast_convert_type(k ^ xm, jnp.float32)
                    plsc.store_scatter(xbuf, [off], f)
                else:
                    plsc.store_scatter(kdst, [off], k)
                plsc.store_scatter(pdst, [off], p)
                plsc.addupdate_scatter(hist, [d], occ, mask=last)

            def scat_body(i, _):
                scat_one(2 * i)
                scat_one(2 * i + 1)
                return 0

            lax.fori_loop(0, nv // 2, scat_body, 0)

        pltpu.sync_copy(xbuf, vals_hbm.at[row])
        pltpu.sync_copy(p0, idx_hbm.at[row])
        return 0

    lax.fori_loop(0, rpw, row_body, 0)


def _mix_rows_body(n, rpw, nch, lam_hbm, pm_hbm, vals_hbm, idx_hbm, out_hbm,
                   lbuf, pbuf, va, vb, ib, ob):
    nv = n // L
    w = _wid()
    b = w // 2
    half = (w % 2) * rpw
    iota = lax.iota(jnp.int32, L)

    pltpu.sync_copy(lam_hbm, lbuf)
    pltpu.sync_copy(pm_hbm, pbuf)
    lam = jnp.sum(jnp.where(iota == b, lbuf[...], 0.0))
    pb = jnp.sum(jnp.where(iota == b, pbuf[...], 0))
    lamv = jnp.full((L,), lam, jnp.float32)
    one_m = jnp.full((L,), 1.0, jnp.float32) - lamv

    def row_body(j, _):
        row = w * rpw + j
        prow = pb * nch + half + j
        pltpu.sync_copy(vals_hbm.at[row], va)
        pltpu.sync_copy(vals_hbm.at[prow], vb)
        pltpu.sync_copy(idx_hbm.at[row], ib)

        def mix_body(i, _):
            s = pl.ds(i * L, L)
            mixed = lamv * va[s] + one_m * vb[s]
            plsc.store_scatter(ob, [ib[s]], mixed)
            return 0

        lax.fori_loop(0, nv, mix_body, 0)
        pltpu.sync_copy(ob, out_hbm.at[row])
        return 0

    lax.fori_loop(0, rpw, row_body, 0)


@jax.jit
def kernel(x, lmda, perm):
    bv, cv, hv, wv = x.shape
    n = hv * wv
    r = bv * cv
    nw = 32
    assert r % nw == 0 and n % (2 * L) == 0
    rpw = r // nw

    xv = x.reshape(r, n)
    lam = lmda.reshape(bv).astype(jnp.float32)
    pm = perm.astype(jnp.int32)

    mesh = plsc.VectorSubcoreMesh(core_axis_name="c", subcore_axis_name="s")

    sort_call = pl.kernel(
        functools.partial(_sort_rows_body, n, rpw),
        out_type=[
            jax.ShapeDtypeStruct((r, n), jnp.float32),
            jax.ShapeDtypeStruct((r, n), jnp.int32),
        ],
        mesh=mesh,
        scratch_types=[
            pltpu.VMEM((n,), jnp.float32),
            pltpu.VMEM((n,), jnp.int32),
            pltpu.VMEM((n,), jnp.int32),
            pltpu.VMEM((n,), jnp.int32),
            pltpu.VMEM((n,), jnp.int32),
            pltpu.VMEM((NPASS * NBUCK,), jnp.int32),
        ],
        compiler_params=pltpu.CompilerParams(needs_layout_passes=False),
    )
    vals, idxs = sort_call(xv)

    mix_call = pl.kernel(
        functools.partial(_mix_rows_body, n, rpw, cv),
        out_type=jax.ShapeDtypeStruct((r, n), jnp.float32),
        mesh=mesh,
        scratch_types=[
            pltpu.VMEM((bv,), jnp.float32),
            pltpu.VMEM((bv,), jnp.int32),
            pltpu.VMEM((n,), jnp.float32),
            pltpu.VMEM((n,), jnp.float32),
            pltpu.VMEM((n,), jnp.int32),
            pltpu.VMEM((n,), jnp.float32),
        ],
        compiler_params=pltpu.CompilerParams(needs_layout_passes=False),
    )
    out = mix_call(lam, pm, vals, idxs)
    return out.reshape(bv, cv, hv, wv)


# speedup vs baseline: 4.6553x; 1.0061x over previous
import functools

import jax
import jax.numpy as jnp
from jax import lax
from jax.experimental import pallas as pl
from jax.experimental.pallas import tpu as pltpu
from jax.experimental.pallas import tpu_sc as plsc

L = 16
NBITS = 8
NBUCK = 1 << NBITS
DMASK = NBUCK - 1
NPASS = 4
TOPBIT = -2147483648


def _wid():
    return lax.axis_index("s") * 2 + lax.axis_index("c")


def _sort_rows_body(n, rpw, x_hbm, vals_hbm, idx_hbm, xbuf, k0, p0, k1, p1, hist):
    nv = n // L
    w = _wid()
    iota = lax.iota(jnp.int32, L)

    def row_body(j, _):
        row = w * rpw + j
        pltpu.sync_copy(x_hbm.at[row], xbuf)

        def zero_body(i, _):
            hist[pl.ds(i * L, L)] = jnp.zeros((L,), jnp.int32)
            return 0

        lax.fori_loop(0, NPASS * NBUCK // L, zero_body, 0)

        def prep_one(i):
            s = pl.ds(i * L, L)
            bits = lax.bitcast_convert_type(xbuf[s], jnp.int32)
            xm = lax.shift_right_arithmetic(bits, 31) | TOPBIT
            k = bits ^ xm
            k0[s] = k
            p0[s] = iota + i * L
            for pss in range(NPASS):
                d = (lax.shift_right_logical(k, pss * NBITS) & DMASK) + pss * NBUCK
                occ, last = plsc.scan_count(d)
                plsc.addupdate_scatter(hist, [d], occ, mask=last)

        def prep_body(i, _):
            prep_one(2 * i)
            prep_one(2 * i + 1)
            return 0

        lax.fori_loop(0, nv // 2, prep_body, 0)

        def scan_body(i, carry):
            s = pl.ds(i * L, L)
            v = hist[s]
            cs = plsc.cumsum(v)
            hist[s] = cs - v + carry
            new = carry + jnp.sum(v)
            return jnp.where((i + 1) % (NBUCK // L) == 0, 0, new)

        lax.fori_loop(0, NPASS * NBUCK // L, scan_body, jnp.int32(0))

        bufs = [(k0, p0, k1, p1), (k1, p1, k0, p0),
                (k0, p0, k1, p1), (k1, p1, k0, p0)]
        for pss, (ksrc, psrc, kdst, pdst) in enumerate(bufs):
            last_pass = pss == NPASS - 1

            def scat_one(i):
                s = pl.ds(i * L, L)
                k = ksrc[s]
                p = psrc[s]
                d = (lax.shift_right_logical(k, pss * NBITS) & DMASK) + pss * NBUCK
                occ, last = plsc.scan_count(d)
                base = plsc.load_gather(hist, [d])
                off = base + occ - 1
                if last_pass:
                    xm = ~lax.shift_right_arithmetic(k, 31) | TOPBIT
                    f = lax.bitcast_convert_type(k ^ xm, jnp.float32)
                    plsc.store_scatter(xbuf, [off], f)
                else:
                    plsc.store_scatter(kdst, [off], k)
                plsc.store_scatter(pdst, [off], p)
                plsc.addupdate_scatter(hist, [d], occ, mask=last)

            def scat_body(i, _):
                scat_one(4 * i)
                scat_one(4 * i + 1)
                scat_one(4 * i + 2)
                scat_one(4 * i + 3)
                return 0

            lax.fori_loop(0, nv // 4, scat_body, 0)

        pltpu.sync_copy(xbuf, vals_hbm.at[row])
        pltpu.sync_copy(p0, idx_hbm.at[row])
        return 0

    lax.fori_loop(0, rpw, row_body, 0)


def _mix_rows_body(n, rpw, nch, lam_hbm, pm_hbm, vals_hbm, idx_hbm, out_hbm,
                   lbuf, pbuf, va, vb, ib, ob):
    nv = n // L
    w = _wid()
    b = w // 2
    half = (w % 2) * rpw
    iota = lax.iota(jnp.int32, L)

    pltpu.sync_copy(lam_hbm, lbuf)
    pltpu.sync_copy(pm_hbm, pbuf)
    lam = jnp.sum(jnp.where(iota == b, lbuf[...], 0.0))
    pb = jnp.sum(jnp.where(iota == b, pbuf[...], 0))
    lamv = jnp.full((L,), lam, jnp.float32)
    one_m = jnp.full((L,), 1.0, jnp.float32) - lamv

    def row_body(j, _):
        row = w * rpw + j
        prow = pb * nch + half + j
        pltpu.sync_copy(vals_hbm.at[row], va)
        pltpu.sync_copy(vals_hbm.at[prow], vb)
        pltpu.sync_copy(idx_hbm.at[row], ib)

        def mix_one(i):
            s = pl.ds(i * L, L)
            mixed = lamv * va[s] + one_m * vb[s]
            plsc.store_scatter(ob, [ib[s]], mixed)

        def mix_body(i, _):
            mix_one(4 * i)
            mix_one(4 * i + 1)
            mix_one(4 * i + 2)
            mix_one(4 * i + 3)
            return 0

        lax.fori_loop(0, nv // 4, mix_body, 0)
        pltpu.sync_copy(ob, out_hbm.at[row])
        return 0

    lax.fori_loop(0, rpw, row_body, 0)


@jax.jit
def kernel(x, lmda, perm):
    bv, cv, hv, wv = x.shape
    n = hv * wv
    r = bv * cv
    nw = 32
    assert r % nw == 0 and n % (2 * L) == 0
    rpw = r // nw

    xv = x.reshape(r, n)
    lam = lmda.reshape(bv).astype(jnp.float32)
    pm = perm.astype(jnp.int32)

    mesh = plsc.VectorSubcoreMesh(core_axis_name="c", subcore_axis_name="s")

    sort_call = pl.kernel(
        functools.partial(_sort_rows_body, n, rpw),
        out_type=[
            jax.ShapeDtypeStruct((r, n), jnp.float32),
            jax.ShapeDtypeStruct((r, n), jnp.int32),
        ],
        mesh=mesh,
        scratch_types=[
            pltpu.VMEM((n,), jnp.float32),
            pltpu.VMEM((n,), jnp.int32),
            pltpu.VMEM((n,), jnp.int32),
            pltpu.VMEM((n,), jnp.int32),
            pltpu.VMEM((n,), jnp.int32),
            pltpu.VMEM((NPASS * NBUCK,), jnp.int32),
        ],
        compiler_params=pltpu.CompilerParams(needs_layout_passes=False),
    )
    vals, idxs = sort_call(xv)

    mix_call = pl.kernel(
        functools.partial(_mix_rows_body, n, rpw, cv),
        out_type=jax.ShapeDtypeStruct((r, n), jnp.float32),
        mesh=mesh,
        scratch_types=[
            pltpu.VMEM((bv,), jnp.float32),
            pltpu.VMEM((bv,), jnp.int32),
            pltpu.VMEM((n,), jnp.float32),
            pltpu.VMEM((n,), jnp.float32),
            pltpu.VMEM((n,), jnp.int32),
            pltpu.VMEM((n,), jnp.float32),
        ],
        compiler_params=pltpu.CompilerParams(needs_layout_passes=False),
    )
    out = mix_call(lam, pm, vals, idxs)
    return out.reshape(bv, cv, hv, wv)


# speedup vs baseline: 6.3131x; 1.3561x over previous
import functools

import jax
import jax.numpy as jnp
from jax import lax
from jax.experimental import pallas as pl
from jax.experimental.pallas import tpu as pltpu
from jax.experimental.pallas import tpu_sc as plsc

L = 16
NBITS = 8
NBUCK = 1 << NBITS
DMASK = NBUCK - 1
NPASS = 4
TOPBIT = -2147483648


def _wid():
    return lax.axis_index("s") * 2 + lax.axis_index("c")


def _sort_rows_body(n, rpw, x_hbm, vals_hbm, idx_hbm, xbuf, k0, p0, k1, p1, hist):
    nv = n // L
    w = _wid()
    iota = lax.iota(jnp.int32, L)

    def row_body(j, _):
        row = w * rpw + j
        pltpu.sync_copy(x_hbm.at[row], xbuf)

        def zero_body(i, _):
            hist[pl.ds(i * L, L)] = jnp.zeros((L,), jnp.int32)
            return 0

        lax.fori_loop(0, NPASS * NBUCK // L, zero_body, 0)

        def prep_one(i):
            s = pl.ds(i * L, L)
            bits = lax.bitcast_convert_type(xbuf[s], jnp.int32)
            xm = lax.shift_right_arithmetic(bits, 31) | TOPBIT
            k = bits ^ xm
            k0[s] = k
            p0[s] = iota + i * L
            for pss in range(NPASS):
                d = (lax.shift_right_logical(k, pss * NBITS) & DMASK) + pss * NBUCK
                occ, last = plsc.scan_count(d)
                plsc.addupdate_scatter(hist, [d], occ, mask=last)

        def prep_body(i, _):
            prep_one(2 * i)
            prep_one(2 * i + 1)
            return 0

        lax.fori_loop(0, nv // 2, prep_body, 0)

        def scan_body(i, carry):
            s = pl.ds(i * L, L)
            v = hist[s]
            cs = plsc.cumsum(v)
            hist[s] = cs - v + carry
            new = carry + jnp.sum(v)
            return jnp.where((i + 1) % (NBUCK // L) == 0, 0, new)

        lax.fori_loop(0, NPASS * NBUCK // L, scan_body, jnp.int32(0))

        bufs = [(k0, p0, k1, p1), (k1, p1, k0, p0),
                (k0, p0, k1, p1), (k1, p1, k0, p0)]
        for pss, (ksrc, psrc, kdst, pdst) in enumerate(bufs):
            last_pass = pss == NPASS - 1

            def fetch(i):
                s = pl.ds(i * L, L)
                k = ksrc[s]
                p = psrc[s]
                d = (lax.shift_right_logical(k, pss * NBITS) & DMASK) + pss * NBUCK
                occ, last = plsc.scan_count(d)
                return k, p, d, occ, last

            def commit(st):
                k, p, d, occ, last = st
                base = plsc.load_gather(hist, [d])
                off = base + occ - 1
                if last_pass:
                    xm = ~lax.shift_right_arithmetic(k, 31) | TOPBIT
                    f = lax.bitcast_convert_type(k ^ xm, jnp.float32)
                    plsc.store_scatter(xbuf, [off], f)
                else:
                    plsc.store_scatter(kdst, [off], k)
                plsc.store_scatter(pdst, [off], p)
                plsc.addupdate_scatter(hist, [d], occ, mask=last)

            def scat_body(i, st):
                nst = fetch(i + 1)
                commit(st)
                return nst

            st = lax.fori_loop(0, nv - 1, scat_body, fetch(0))
            commit(st)

        pltpu.sync_copy(xbuf, vals_hbm.at[row])
        pltpu.sync_copy(p0, idx_hbm.at[row])
        return 0

    lax.fori_loop(0, rpw, row_body, 0)


def _mix_rows_body(n, rpw, nch, lam_hbm, pm_hbm, vals_hbm, idx_hbm, out_hbm,
                   lbuf, pbuf, va, vb, ib, ob):
    nv = n // L
    w = _wid()
    b = w // 2
    half = (w % 2) * rpw
    iota = lax.iota(jnp.int32, L)

    pltpu.sync_copy(lam_hbm, lbuf)
    pltpu.sync_copy(pm_hbm, pbuf)
    lam = jnp.sum(jnp.where(iota == b, lbuf[...], 0.0))
    pb = jnp.sum(jnp.where(iota == b, pbuf[...], 0))
    lamv = jnp.full((L,), lam, jnp.float32)
    one_m = jnp.full((L,), 1.0, jnp.float32) - lamv

    def row_body(j, _):
        row = w * rpw + j
        prow = pb * nch + half + j
        pltpu.sync_copy(vals_hbm.at[row], va)
        pltpu.sync_copy(vals_hbm.at[prow], vb)
        pltpu.sync_copy(idx_hbm.at[row], ib)

        def mfetch(i):
            s = pl.ds(i * L, L)
            return va[s], vb[s], ib[s]

        def mcommit(st):
            a, bb, ii = st
            plsc.store_scatter(ob, [ii], lamv * a + one_m * bb)

        def mix_body(i, st):
            nst = mfetch(i + 1)
            mcommit(st)
            return nst

        st = lax.fori_loop(0, nv - 1, mix_body, mfetch(0))
        mcommit(st)
        pltpu.sync_copy(ob, out_hbm.at[row])
        return 0

    lax.fori_loop(0, rpw, row_body, 0)


@jax.jit
def kernel(x, lmda, perm):
    bv, cv, hv, wv = x.shape
    n = hv * wv
    r = bv * cv
    nw = 32
    assert r % nw == 0 and n % (2 * L) == 0
    rpw = r // nw

    xv = x.reshape(r, n)
    lam = lmda.reshape(bv).astype(jnp.float32)
    pm = perm.astype(jnp.int32)

    mesh = plsc.VectorSubcoreMesh(core_axis_name="c", subcore_axis_name="s")

    sort_call = pl.kernel(
        functools.partial(_sort_rows_body, n, rpw),
        out_type=[
            jax.ShapeDtypeStruct((r, n), jnp.float32),
            jax.ShapeDtypeStruct((r, n), jnp.int32),
        ],
        mesh=mesh,
        scratch_types=[
            pltpu.VMEM((n,), jnp.float32),
            pltpu.VMEM((n,), jnp.int32),
            pltpu.VMEM((n,), jnp.int32),
            pltpu.VMEM((n,), jnp.int32),
            pltpu.VMEM((n,), jnp.int32),
            pltpu.VMEM((NPASS * NBUCK,), jnp.int32),
        ],
        compiler_params=pltpu.CompilerParams(needs_layout_passes=False),
    )
    vals, idxs = sort_call(xv)

    mix_call = pl.kernel(
        functools.partial(_mix_rows_body, n, rpw, cv),
        out_type=jax.ShapeDtypeStruct((r, n), jnp.float32),
        mesh=mesh,
        scratch_types=[
            pltpu.VMEM((bv,), jnp.float32),
            pltpu.VMEM((bv,), jnp.int32),
            pltpu.VMEM((n,), jnp.float32),
            pltpu.VMEM((n,), jnp.float32),
            pltpu.VMEM((n,), jnp.int32),
            pltpu.VMEM((n,), jnp.float32),
        ],
        compiler_params=pltpu.CompilerParams(needs_layout_passes=False),
    )
    out = mix_call(lam, pm, vals, idxs)
    return out.reshape(bv, cv, hv, wv)


# speedup vs baseline: 7.1294x; 1.1293x over previous
import functools

import jax
import jax.numpy as jnp
from jax import lax
from jax.experimental import pallas as pl
from jax.experimental.pallas import tpu as pltpu
from jax.experimental.pallas import tpu_sc as plsc

L = 16
NBITS = 8
NBUCK = 1 << NBITS
DMASK = NBUCK - 1
NPASS = 4
TOPBIT = -2147483648


def _wid():
    return lax.axis_index("s") * 2 + lax.axis_index("c")


def _sort_rows_body(n, rpw, x_hbm, vals_hbm, idx_hbm, xbuf, k0, p0, k1, p1, hist):
    nv = n // L
    w = _wid()
    iota = lax.iota(jnp.int32, L)

    def row_body(j, _):
        row = w * rpw + j
        pltpu.sync_copy(x_hbm.at[row], xbuf)

        def zero_body(i, _):
            hist[pl.ds(i * L, L)] = jnp.zeros((L,), jnp.int32)
            return 0

        lax.fori_loop(0, NPASS * NBUCK // L, zero_body, 0)

        def prep_one(i):
            s = pl.ds(i * L, L)
            bits = lax.bitcast_convert_type(xbuf[s], jnp.int32)
            xm = lax.shift_right_arithmetic(bits, 31) | TOPBIT
            k = bits ^ xm
            k0[s] = k
            p0[s] = iota + i * L
            for pss in range(NPASS):
                d = (lax.shift_right_logical(k, pss * NBITS) & DMASK) + pss * NBUCK
                occ, last = plsc.scan_count(d)
                plsc.addupdate_scatter(hist, [d], occ, mask=last)

        def prep_body(i, _):
            prep_one(2 * i)
            prep_one(2 * i + 1)
            return 0

        lax.fori_loop(0, nv // 2, prep_body, 0)

        def scan_body(i, carry):
            s = pl.ds(i * L, L)
            v = hist[s]
            cs = plsc.cumsum(v)
            hist[s] = cs - v + carry
            new = carry + jnp.sum(v)
            return jnp.where((i + 1) % (NBUCK // L) == 0, 0, new)

        lax.fori_loop(0, NPASS * NBUCK // L, scan_body, jnp.int32(0))

        bufs = [(k0, p0, k1, p1), (k1, p1, k0, p0),
                (k0, p0, k1, p1), (k1, p1, k0, p0)]
        for pss, (ksrc, psrc, kdst, pdst) in enumerate(bufs):
            last_pass = pss == NPASS - 1

            def fetch(i):
                s = pl.ds(i * L, L)
                k = ksrc[s]
                p = psrc[s]
                d = (lax.shift_right_logical(k, pss * NBITS) & DMASK) + pss * NBUCK
                occ, last = plsc.scan_count(d)
                return k, p, d, occ, last

            def commit(st):
                k, p, d, occ, last = st
                base = plsc.load_gather(hist, [d])
                off = base + occ - 1
                if last_pass:
                    xm = ~lax.shift_right_arithmetic(k, 31) | TOPBIT
                    f = lax.bitcast_convert_type(k ^ xm, jnp.float32)
                    plsc.store_scatter(xbuf, [off], f)
                else:
                    plsc.store_scatter(kdst, [off], k)
                plsc.store_scatter(pdst, [off], p)
                plsc.addupdate_scatter(hist, [d], occ, mask=last)

            def scat_body(i, st):
                nst = fetch(i + 1)
                commit(st)
                return nst

            st = lax.fori_loop(0, nv - 1, scat_body, fetch(0))
            commit(st)

        pltpu.sync_copy(xbuf, vals_hbm.at[row])
        pltpu.sync_copy(p0, idx_hbm.at[row])
        return 0

    lax.fori_loop(0, rpw, row_body, 0)


def _mix_rows_body(n, rpw, nch, lam_hbm, pm_hbm, vals_hbm, idx_hbm, out_hbm,
                   lbuf, pbuf, va0, va1, vb0, vb1, ib0, ib1, ob0, ob1,
                   sem_in, sem_out):
    nv = n // L
    w = _wid()
    b = w // 2
    half = (w % 2) * rpw
    iota = lax.iota(jnp.int32, L)
    vas, vbs, ibs, obs = (va0, va1), (vb0, vb1), (ib0, ib1), (ob0, ob1)

    pltpu.sync_copy(lam_hbm, lbuf)
    pltpu.sync_copy(pm_hbm, pbuf)
    lam = jnp.sum(jnp.where(iota == b, lbuf[...], 0.0))
    pb = jnp.sum(jnp.where(iota == b, pbuf[...], 0))
    lamv = jnp.full((L,), lam, jnp.float32)
    one_m = jnp.full((L,), 1.0, jnp.float32) - lamv
    row0 = w * rpw
    prow0 = pb * nch + half

    def start_in(j, va, vb, ib):
        pltpu.async_copy(vals_hbm.at[row0 + j], va, sem_in)
        pltpu.async_copy(vals_hbm.at[prow0 + j], vb, sem_in)
        pltpu.async_copy(idx_hbm.at[row0 + j], ib, sem_in)

    def wait_in(j, va, vb, ib):
        pltpu.make_async_copy(vals_hbm.at[row0 + j], va, sem_in).wait()
        pltpu.make_async_copy(vals_hbm.at[prow0 + j], vb, sem_in).wait()
        pltpu.make_async_copy(idx_hbm.at[row0 + j], ib, sem_in).wait()

    start_in(0, vas[0], vbs[0], ibs[0])

    def do_row(j, cur, prefetch_ok):
        va, vb, ib, ob = vas[cur], vbs[cur], ibs[cur], obs[cur]
        nva, nvb, nib = vas[1 - cur], vbs[1 - cur], ibs[1 - cur]
        wait_in(j, va, vb, ib)

        @pl.when(prefetch_ok)
        def _():
            start_in(j + 1, nva, nvb, nib)

        @pl.when(j >= 2)
        def _():
            pltpu.make_async_copy(ob, out_hbm.at[row0 + j - 2], sem_out).wait()

        def mfetch(i):
            s = pl.ds(i * L, L)
            return va[s], vb[s], ib[s]

        def mcommit(st):
            a, bb, ii = st
            plsc.store_scatter(ob, [ii], lamv * a + one_m * bb)

        def mix_body(i, st):
            nst = mfetch(i + 1)
            mcommit(st)
            return nst

        st = lax.fori_loop(0, nv - 1, mix_body, mfetch(0))
        mcommit(st)
        pltpu.async_copy(ob, out_hbm.at[row0 + j], sem_out)

    def pair_body(jj, _):
        j0 = 2 * jj
        do_row(j0, 0, j0 + 1 < rpw)
        do_row(j0 + 1, 1, j0 + 2 < rpw)
        return 0

    lax.fori_loop(0, rpw // 2, pair_body, 0)
    pltpu.make_async_copy(obs[rpw % 2], out_hbm.at[row0 + rpw - 2], sem_out).wait()
    pltpu.make_async_copy(obs[1 - rpw % 2], out_hbm.at[row0 + rpw - 1], sem_out).wait()


@jax.jit
def kernel(x, lmda, perm):
    bv, cv, hv, wv = x.shape
    n = hv * wv
    r = bv * cv
    nw = 32
    assert r % nw == 0 and n % (2 * L) == 0
    rpw = r // nw

    xv = x.reshape(r, n)
    lam = lmda.reshape(bv).astype(jnp.float32)
    pm = perm.astype(jnp.int32)

    mesh = plsc.VectorSubcoreMesh(core_axis_name="c", subcore_axis_name="s")

    sort_call = pl.kernel(
        functools.partial(_sort_rows_body, n, rpw),
        out_type=[
            jax.ShapeDtypeStruct((r, n), jnp.float32),
            jax.ShapeDtypeStruct((r, n), jnp.int32),
        ],
        mesh=mesh,
        scratch_types=[
            pltpu.VMEM((n,), jnp.float32),
            pltpu.VMEM((n,), jnp.int32),
            pltpu.VMEM((n,), jnp.int32),
            pltpu.VMEM((n,), jnp.int32),
            pltpu.VMEM((n,), jnp.int32),
            pltpu.VMEM((NPASS * NBUCK,), jnp.int32),
        ],
        compiler_params=pltpu.CompilerParams(needs_layout_passes=False),
    )
    vals, idxs = sort_call(xv)

    mix_call = pl.kernel(
        functools.partial(_mix_rows_body, n, rpw, cv),
        out_type=jax.ShapeDtypeStruct((r, n), jnp.float32),
        mesh=mesh,
        scratch_types=[
            pltpu.VMEM((bv,), jnp.float32),
            pltpu.VMEM((bv,), jnp.int32),
            pltpu.VMEM((n,), jnp.float32),
            pltpu.VMEM((n,), jnp.float32),
            pltpu.VMEM((n,), jnp.float32),
            pltpu.VMEM((n,), jnp.float32),
            pltpu.VMEM((n,), jnp.int32),
            pltpu.VMEM((n,), jnp.int32),
            pltpu.VMEM((n,), jnp.float32),
            pltpu.VMEM((n,), jnp.float32),
            pltpu.SemaphoreType.DMA,
            pltpu.SemaphoreType.DMA,
        ],
        compiler_params=pltpu.CompilerParams(needs_layout_passes=False),
    )
    out = mix_call(lam, pm, vals, idxs)
    return out.reshape(bv, cv, hv, wv)


# speedup vs baseline: 8.5655x; 1.2014x over previous
import functools

import jax
import jax.numpy as jnp
from jax import lax
from jax.experimental import pallas as pl
from jax.experimental.pallas import tpu as pltpu
from jax.experimental.pallas import tpu_sc as plsc

L = 16
NBITS = 8
NBUCK = 1 << NBITS
DMASK = NBUCK - 1
NPASS = 4
TOPBIT = -2147483648


def _wid():
    return lax.axis_index("s") * 2 + lax.axis_index("c")


def _sort_rows_body(n, rpw, x_hbm, vals_hbm, idx_hbm,
                    xb0, xb1, xb2, k0, p0, k1, p1, pout, hist,
                    sem_in, sem_v, sem_i):
    nv = n // L
    w = _wid()
    iota = lax.iota(jnp.int32, L)
    xbufs = (xb0, xb1, xb2)
    row0 = w * rpw

    pltpu.async_copy(x_hbm.at[row0], xb0, sem_in)

    def do_row(j, xbuf, nxt):
        row = row0 + j
        pltpu.make_async_copy(x_hbm.at[row], xbuf, sem_in).wait()

        @pl.when(j >= 2)
        def _():
            pltpu.make_async_copy(nxt, vals_hbm.at[row - 2], sem_v).wait()

        @pl.when(j + 1 < rpw)
        def _():
            pltpu.async_copy(x_hbm.at[row + 1], nxt, sem_in)

        def zero_body(i, _):
            hist[pl.ds(i * L, L)] = jnp.zeros((L,), jnp.int32)
            return 0

        lax.fori_loop(0, NPASS * NBUCK // L, zero_body, 0)

        def pfetch(i):
            bits = lax.bitcast_convert_type(xbuf[pl.ds(i * L, L)], jnp.int32)
            xm = lax.shift_right_arithmetic(bits, 31) | TOPBIT
            return bits ^ xm

        def pcommit(i, k):
            s = pl.ds(i * L, L)
            k0[s] = k
            p0[s] = iota + i * L
            for pss in range(NPASS):
                d = (lax.shift_right_logical(k, pss * NBITS) & DMASK) + pss * NBUCK
                occ, last = plsc.scan_count(d)
                plsc.addupdate_scatter(hist, [d], occ, mask=last)

        def prep_body(i, k):
            kn = pfetch(i + 1)
            pcommit(i, k)
            return kn

        k_last = lax.fori_loop(0, nv - 1, prep_body, pfetch(0))
        pcommit(nv - 1, k_last)

        def scan_body(i, carry):
            s = pl.ds(i * L, L)
            v = hist[s]
            cs = plsc.cumsum(v)
            hist[s] = cs - v + carry
            new = carry + jnp.sum(v)
            return jnp.where((i + 1) % (NBUCK // L) == 0, 0, new)

        lax.fori_loop(0, NPASS * NBUCK // L, scan_body, jnp.int32(0))

        bufs = [(k0, p0, k1, p1), (k1, p1, k0, p0),
                (k0, p0, k1, p1), (k1, p1, k0, pout)]
        for pss, (ksrc, psrc, kdst, pdst) in enumerate(bufs):
            last_pass = pss == NPASS - 1
            if last_pass:
                @pl.when(j >= 1)
                def _():
                    pltpu.make_async_copy(pout, idx_hbm.at[row - 1], sem_i).wait()

            def fetch(i):
                s = pl.ds(i * L, L)
                k = ksrc[s]
                p = psrc[s]
                d = (lax.shift_right_logical(k, pss * NBITS) & DMASK) + pss * NBUCK
                occ, last = plsc.scan_count(d)
                return k, p, d, occ, last

            def commit(st):
                k, p, d, occ, last = st
                base = plsc.load_gather(hist, [d])
                off = base + occ - 1
                if last_pass:
                    xm = ~lax.shift_right_arithmetic(k, 31) | TOPBIT
                    f = lax.bitcast_convert_type(k ^ xm, jnp.float32)
                    plsc.store_scatter(xbuf, [off], f)
                else:
                    plsc.store_scatter(kdst, [off], k)
                plsc.store_scatter(pdst, [off], p)
                plsc.addupdate_scatter(hist, [d], occ, mask=last)

            def scat_body(i, st):
                nst = fetch(i + 1)
                commit(st)
                return nst

            st = lax.fori_loop(0, nv - 1, scat_body, fetch(0))
            commit(st)

        pltpu.async_copy(xbuf, vals_hbm.at[row], sem_v)
        pltpu.async_copy(pout, idx_hbm.at[row], sem_i)

    def tri_body(jj, _):
        j0 = 3 * jj
        do_row(j0, xbufs[0], xbufs[1])
        do_row(j0 + 1, xbufs[1], xbufs[2])
        do_row(j0 + 2, xbufs[2], xbufs[0])
        return 0

    lax.fori_loop(0, rpw // 3, tri_body, 0)
    pltpu.make_async_copy(xbufs[1], vals_hbm.at[row0 + rpw - 2], sem_v).wait()
    pltpu.make_async_copy(xbufs[2], vals_hbm.at[row0 + rpw - 1], sem_v).wait()
    pltpu.make_async_copy(pout, idx_hbm.at[row0 + rpw - 1], sem_i).wait()


def _mix_rows_body(n, rpw, nch, lam_hbm, pm_hbm, vals_hbm, idx_hbm, out_hbm,
                   lbuf, pbuf, va0, va1, vb0, vb1, ib0, ib1, ob0, ob1,
                   sem_in, sem_out):
    nv = n // L
    w = _wid()
    b = w // 2
    half = (w % 2) * rpw
    iota = lax.iota(jnp.int32, L)
    vas, vbs, ibs, obs = (va0, va1), (vb0, vb1), (ib0, ib1), (ob0, ob1)

    pltpu.sync_copy(lam_hbm, lbuf)
    pltpu.sync_copy(pm_hbm, pbuf)
    lam = jnp.sum(jnp.where(iota == b, lbuf[...], 0.0))
    pb = jnp.sum(jnp.where(iota == b, pbuf[...], 0))
    lamv = jnp.full((L,), lam, jnp.float32)
    one_m = jnp.full((L,), 1.0, jnp.float32) - lamv
    row0 = w * rpw
    prow0 = pb * nch + half

    def start_in(j, va, vb, ib):
        pltpu.async_copy(vals_hbm.at[row0 + j], va, sem_in)
        pltpu.async_copy(vals_hbm.at[prow0 + j], vb, sem_in)
        pltpu.async_copy(idx_hbm.at[row0 + j], ib, sem_in)

    def wait_in(j, va, vb, ib):
        pltpu.make_async_copy(vals_hbm.at[row0 + j], va, sem_in).wait()
        pltpu.make_async_copy(vals_hbm.at[prow0 + j], vb, sem_in).wait()
        pltpu.make_async_copy(idx_hbm.at[row0 + j], ib, sem_in).wait()

    start_in(0, vas[0], vbs[0], ibs[0])

    def do_row(j, cur, prefetch_ok):
        va, vb, ib, ob = vas[cur], vbs[cur], ibs[cur], obs[cur]
        nva, nvb, nib = vas[1 - cur], vbs[1 - cur], ibs[1 - cur]
        wait_in(j, va, vb, ib)

        @pl.when(prefetch_ok)
        def _():
            start_in(j + 1, nva, nvb, nib)

        @pl.when(j >= 2)
        def _():
            pltpu.make_async_copy(ob, out_hbm.at[row0 + j - 2], sem_out).wait()

        def mfetch(i):
            s = pl.ds(i * L, L)
            return va[s], vb[s], ib[s]

        def mcommit(st):
            a, bb, ii = st
            plsc.store_scatter(ob, [ii], lamv * a + one_m * bb)

        def mix_body(i, st):
            nst = mfetch(i + 1)
            mcommit(st)
            return nst

        st = lax.fori_loop(0, nv - 1, mix_body, mfetch(0))
        mcommit(st)
        pltpu.async_copy(ob, out_hbm.at[row0 + j], sem_out)

    def pair_body(jj, _):
        j0 = 2 * jj
        do_row(j0, 0, j0 + 1 < rpw)
        do_row(j0 + 1, 1, j0 + 2 < rpw)
        return 0

    lax.fori_loop(0, rpw // 2, pair_body, 0)
    pltpu.make_async_copy(obs[rpw % 2], out_hbm.at[row0 + rpw - 2], sem_out).wait()
    pltpu.make_async_copy(obs[1 - rpw % 2], out_hbm.at[row0 + rpw - 1], sem_out).wait()


@jax.jit
def kernel(x, lmda, perm):
    bv, cv, hv, wv = x.shape
    n = hv * wv
    r = bv * cv
    nw = 32
    rpw = r // nw
    assert r % nw == 0 and n % (2 * L) == 0 and rpw % 6 == 0

    xv = x.reshape(r, n)
    lam = lmda.reshape(bv).astype(jnp.float32)
    pm = perm.astype(jnp.int32)

    mesh = plsc.VectorSubcoreMesh(core_axis_name="c", subcore_axis_name="s")

    sort_call = pl.kernel(
        functools.partial(_sort_rows_body, n, rpw),
        out_type=[
            jax.ShapeDtypeStruct((r, n), jnp.float32),
            jax.ShapeDtypeStruct((r, n), jnp.int32),
        ],
        mesh=mesh,
        scratch_types=[
            pltpu.VMEM((n,), jnp.float32),
            pltpu.VMEM((n,), jnp.float32),
            pltpu.VMEM((n,), jnp.float32),
            pltpu.VMEM((n,), jnp.int32),
            pltpu.VMEM((n,), jnp.int32),
            pltpu.VMEM((n,), jnp.int32),
            pltpu.VMEM((n,), jnp.int32),
            pltpu.VMEM((n,), jnp.int32),
            pltpu.VMEM((NPASS * NBUCK,), jnp.int32),
            pltpu.SemaphoreType.DMA,
            pltpu.SemaphoreType.DMA,
            pltpu.SemaphoreType.DMA,
        ],
        compiler_params=pltpu.CompilerParams(needs_layout_passes=False),
    )
    vals, idxs = sort_call(xv)

    mix_call = pl.kernel(
        functools.partial(_mix_rows_body, n, rpw, cv),
        out_type=jax.ShapeDtypeStruct((r, n), jnp.float32),
        mesh=mesh,
        scratch_types=[
            pltpu.VMEM((bv,), jnp.float32),
            pltpu.VMEM((bv,), jnp.int32),
            pltpu.VMEM((n,), jnp.float32),
            pltpu.VMEM((n,), jnp.float32),
            pltpu.VMEM((n,), jnp.float32),
            pltpu.VMEM((n,), jnp.float32),
            pltpu.VMEM((n,), jnp.int32),
            pltpu.VMEM((n,), jnp.int32),
            pltpu.VMEM((n,), jnp.float32),
            pltpu.VMEM((n,), jnp.float32),
            pltpu.SemaphoreType.DMA,
            pltpu.SemaphoreType.DMA,
        ],
        compiler_params=pltpu.CompilerParams(needs_layout_passes=False),
    )
    out = mix_call(lam, pm, vals, idxs)
    return out.reshape(bv, cv, hv, wv)


# speedup vs baseline: 9.1816x; 1.0719x over previous
import functools

import jax
import jax.numpy as jnp
from jax import lax
from jax.experimental import pallas as pl
from jax.experimental.pallas import tpu as pltpu
from jax.experimental.pallas import tpu_sc as plsc

L = 16
NBITS = 8
NBUCK = 1 << NBITS
DMASK = NBUCK - 1
NPASS = 4
TOPBIT = -2147483648


def _wid():
    return lax.axis_index("s") * 2 + lax.axis_index("c")


def _sort_rows_body(n, rpw, x_hbm, vals_hbm, idx_hbm,
                    xb0, xb1, xb2, ka0, pa0, ka1, pa1,
                    kb0, pb0, kb1, pb1, pout, hist,
                    sem_in, sem_v, sem_i):
    nv = n // L
    n2 = 2 * n
    hoff = NPASS * NBUCK
    w = _wid()
    iota = lax.iota(jnp.int32, L)
    xbufs = (xb0, xb1, xb2)
    npair = rpw // 2
    base0 = w * rpw * n

    pltpu.async_copy(x_hbm.at[pl.ds(base0, n2)], xb0, sem_in)

    def do_pair(q, xbuf, nxt):
        off = base0 + q * n2
        pltpu.make_async_copy(x_hbm.at[pl.ds(off, n2)], xbuf, sem_in).wait()

        @pl.when(q >= 2)
        def _():
            pltpu.make_async_copy(
                nxt, vals_hbm.at[pl.ds(off - 2 * n2, n2)], sem_v).wait()

        @pl.when(q + 1 < npair)
        def _():
            pltpu.async_copy(x_hbm.at[pl.ds(off + n2, n2)], nxt, sem_in)

        def zero_body(i, _):
            hist[pl.ds(i * L, L)] = jnp.zeros((L,), jnp.int32)
            return 0

        lax.fori_loop(0, 2 * NPASS * NBUCK // L, zero_body, 0)

        def pfetch(i, roff):
            bits = lax.bitcast_convert_type(xbuf[pl.ds(roff + i * L, L)], jnp.int32)
            xm = lax.shift_right_arithmetic(bits, 31) | TOPBIT
            return bits ^ xm

        def pcommit(i, k, kref, pref, ho):
            s = pl.ds(i * L, L)
            kref[s] = k
            pref[s] = iota + i * L
            for pss in range(NPASS):
                d = (lax.shift_right_logical(k, pss * NBITS) & DMASK) + (pss * NBUCK + ho)
                occ, last = plsc.scan_count(d)
                plsc.addupdate_scatter(hist, [d], occ, mask=last)

        def prep_body(i, st):
            ka, kb = st
            kan = pfetch(i + 1, 0)
            kbn = pfetch(i + 1, n)
            pcommit(i, ka, ka0, pa0, 0)
            pcommit(i, kb, kb0, pb0, hoff)
            return (kan, kbn)

        ka, kb = lax.fori_loop(0, nv - 1, prep_body, (pfetch(0, 0), pfetch(0, n)))
        pcommit(nv - 1, ka, ka0, pa0, 0)
        pcommit(nv - 1, kb, kb0, pb0, hoff)

        def scan_body(i, carry):
            s = pl.ds(i * L, L)
            v = hist[s]
            cs = plsc.cumsum(v)
            hist[s] = cs - v + carry
            new = carry + jnp.sum(v)
            return jnp.where((i + 1) % (NBUCK // L) == 0, 0, new)

        lax.fori_loop(0, 2 * NPASS * NBUCK // L, scan_body, jnp.int32(0))

        abufs = [(ka0, pa0, ka1, pa1), (ka1, pa1, ka0, pa0),
                 (ka0, pa0, ka1, pa1), (ka1, pa1, ka0, pa0)]
        bbufs = [(kb0, pb0, kb1, pb1), (kb1, pb1, kb0, pb0),
                 (kb0, pb0, kb1, pb1), (kb1, pb1, kb0, pb0)]
        for pss in range(NPASS):
            last_pass = pss == NPASS - 1
            if last_pass:
                @pl.when(q >= 1)
                def _():
                    pltpu.make_async_copy(
                        pout, idx_hbm.at[pl.ds(off - n2, n2)], sem_i).wait()

            def make_pipe(bufs, ho, roff):
                ksrc, psrc, kdst, pdst = bufs[pss]

                def fetch(i):
                    s = pl.ds(i * L, L)
                    k = ksrc[s]
                    p = psrc[s]
                    d = (lax.shift_right_logical(k, pss * NBITS) & DMASK) + (pss * NBUCK + ho)
                    occ, last = plsc.scan_count(d)
                    return k, p, d, occ, last

                def commit(st):
                    k, p, d, occ, last = st
                    base = plsc.load_gather(hist, [d])
                    o = base + occ - 1
                    if last_pass:
                        xm = ~lax.shift_right_arithmetic(k, 31) | TOPBIT
                        f = lax.bitcast_convert_type(k ^ xm, jnp.float32)
                        plsc.store_scatter(xbuf, [o + roff], f)
                        plsc.store_scatter(pout, [o + roff], p)
                    else:
                        plsc.store_scatter(kdst, [o], k)
                        plsc.store_scatter(pdst, [o], p)
                    plsc.addupdate_scatter(hist, [d], occ, mask=last)

                return fetch, commit

            fa, ca = make_pipe(abufs, 0, 0)
            fb, cb = make_pipe(bbufs, hoff, n)

            def scat_body(i, st):
                sta, stb = st
                na = fa(i + 1)
                nb = fb(i + 1)
                ca(sta)
                cb(stb)
                return (na, nb)

            sta, stb = lax.fori_loop(0, nv - 1, scat_body, (fa(0), fb(0)))
            ca(sta)
            cb(stb)

        pltpu.async_copy(xbuf, vals_hbm.at[pl.ds(off, n2)], sem_v)
        pltpu.async_copy(pout, idx_hbm.at[pl.ds(off, n2)], sem_i)

    def tri_body(jj, _):
        q0 = 3 * jj
        do_pair(q0, xbufs[0], xbufs[1])
        do_pair(q0 + 1, xbufs[1], xbufs[2])
        do_pair(q0 + 2, xbufs[2], xbufs[0])
        return 0

    lax.fori_loop(0, npair // 3, tri_body, 0)
    endoff = base0 + npair * n2
    pltpu.make_async_copy(xbufs[1], vals_hbm.at[pl.ds(endoff - 2 * n2, n2)], sem_v).wait()
    pltpu.make_async_copy(xbufs[2], vals_hbm.at[pl.ds(endoff - n2, n2)], sem_v).wait()
    pltpu.make_async_copy(pout, idx_hbm.at[pl.ds(endoff - n2, n2)], sem_i).wait()


def _mix_rows_body(n, rpw, nch, lam_hbm, pm_hbm, vals_hbm, idx_hbm, out_hbm,
                   lbuf, pbuf, va0, va1, vb0, vb1, ib0, ib1, ob0, ob1,
                   sem_in, sem_out):
    nv = n // L
    w = _wid()
    b = w // 2
    half = (w % 2) * rpw
    iota = lax.iota(jnp.int32, L)
    vas, vbs, ibs, obs = (va0, va1), (vb0, vb1), (ib0, ib1), (ob0, ob1)

    pltpu.sync_copy(lam_hbm, lbuf)
    pltpu.sync_copy(pm_hbm, pbuf)
    lam = jnp.sum(jnp.where(iota == b, lbuf[...], 0.0))
    pb = jnp.sum(jnp.where(iota == b, pbuf[...], 0))
    lamv = jnp.full((L,), lam, jnp.float32)
    one_m = jnp.full((L,), 1.0, jnp.float32) - lamv
    row0 = w * rpw
    prow0 = pb * nch + half

    def start_in(j, va, vb, ib):
        pltpu.async_copy(vals_hbm.at[pl.ds((row0 + j) * n, n)], va, sem_in)
        pltpu.async_copy(vals_hbm.at[pl.ds((prow0 + j) * n, n)], vb, sem_in)
        pltpu.async_copy(idx_hbm.at[pl.ds((row0 + j) * n, n)], ib, sem_in)

    def wait_in(j, va, vb, ib):
        pltpu.make_async_copy(vals_hbm.at[pl.ds((row0 + j) * n, n)], va, sem_in).wait()
        pltpu.make_async_copy(vals_hbm.at[pl.ds((prow0 + j) * n, n)], vb, sem_in).wait()
        pltpu.make_async_copy(idx_hbm.at[pl.ds((row0 + j) * n, n)], ib, sem_in).wait()

    start_in(0, vas[0], vbs[0], ibs[0])

    def do_row(j, cur, prefetch_ok):
        va, vb, ib, ob = vas[cur], vbs[cur], ibs[cur], obs[cur]
        nva, nvb, nib = vas[1 - cur], vbs[1 - cur], ibs[1 - cur]
        wait_in(j, va, vb, ib)

        @pl.when(prefetch_ok)
        def _():
            start_in(j + 1, nva, nvb, nib)

        @pl.when(j >= 2)
        def _():
            pltpu.make_async_copy(ob, out_hbm.at[pl.ds((row0 + j - 2) * n, n)], sem_out).wait()

        def mfetch(i):
            s = pl.ds(i * L, L)
            return va[s], vb[s], ib[s]

        def mcommit(st):
            a, bb, ii = st
            plsc.store_scatter(ob, [ii], lamv * a + one_m * bb)

        def mix_body(i, st):
            nst = mfetch(i + 1)
            mcommit(st)
            return nst

        st = lax.fori_loop(0, nv - 1, mix_body, mfetch(0))
        mcommit(st)
        pltpu.async_copy(ob, out_hbm.at[pl.ds((row0 + j) * n, n)], sem_out)

    def pair_body(jj, _):
        j0 = 2 * jj
        do_row(j0, 0, j0 + 1 < rpw)
        do_row(j0 + 1, 1, j0 + 2 < rpw)
        return 0

    lax.fori_loop(0, rpw // 2, pair_body, 0)
    pltpu.make_async_copy(obs[rpw % 2], out_hbm.at[pl.ds((row0 + rpw - 2) * n, n)], sem_out).wait()
    pltpu.make_async_copy(obs[1 - rpw % 2], out_hbm.at[pl.ds((row0 + rpw - 1) * n, n)], sem_out).wait()


@jax.jit
def kernel(x, lmda, perm):
    bv, cv, hv, wv = x.shape
    n = hv * wv
    r = bv * cv
    nw = 32
    rpw = r // nw
    assert r % nw == 0 and n % (2 * L) == 0 and rpw % 6 == 0

    xv = x.reshape(r * n)
    lam = lmda.reshape(bv).astype(jnp.float32)
    pm = perm.astype(jnp.int32)

    mesh = plsc.VectorSubcoreMesh(core_axis_name="c", subcore_axis_name="s")

    sort_call = pl.kernel(
        functools.partial(_sort_rows_body, n, rpw),
        out_type=[
            jax.ShapeDtypeStruct((r * n,), jnp.float32),
            jax.ShapeDtypeStruct((r * n,), jnp.int32),
        ],
        mesh=mesh,
        scratch_types=[
            pltpu.VMEM((2 * n,), jnp.float32),
            pltpu.VMEM((2 * n,), jnp.float32),
            pltpu.VMEM((2 * n,), jnp.float32),
            pltpu.VMEM((n,), jnp.int32),
            pltpu.VMEM((n,), jnp.int32),
            pltpu.VMEM((n,), jnp.int32),
            pltpu.VMEM((n,), jnp.int32),
            pltpu.VMEM((n,), jnp.int32),
            pltpu.VMEM((n,), jnp.int32),
            pltpu.VMEM((n,), jnp.int32),
            pltpu.VMEM((n,), jnp.int32),
            pltpu.VMEM((2 * n,), jnp.int32),
            pltpu.VMEM((2 * NPASS * NBUCK,), jnp.int32),
            pltpu.SemaphoreType.DMA,
            pltpu.SemaphoreType.DMA,
            pltpu.SemaphoreType.DMA,
        ],
        compiler_params=pltpu.CompilerParams(needs_layout_passes=False),
    )
    vals, idxs = sort_call(xv)

    mix_call = pl.kernel(
        functools.partial(_mix_rows_body, n, rpw, cv),
        out_type=jax.ShapeDtypeStruct((r * n,), jnp.float32),
        mesh=mesh,
        scratch_types=[
            pltpu.VMEM((bv,), jnp.float32),
            pltpu.VMEM((bv,), jnp.int32),
            pltpu.VMEM((n,), jnp.float32),
            pltpu.VMEM((n,), jnp.float32),
            pltpu.VMEM((n,), jnp.float32),
            pltpu.VMEM((n,), jnp.float32),
            pltpu.VMEM((n,), jnp.int32),
            pltpu.VMEM((n,), jnp.int32),
            pltpu.VMEM((n,), jnp.float32),
            pltpu.VMEM((n,), jnp.float32),
            pltpu.SemaphoreType.DMA,
            pltpu.SemaphoreType.DMA,
        ],
        compiler_params=pltpu.CompilerParams(needs_layout_passes=False),
    )
    out = mix_call(lam, pm, vals, idxs)
    return out.reshape(bv, cv, hv, wv)
